# baseline (speedup 1.0000x reference)
import jax
import jax.numpy as jnp
from jax import lax
from jax.experimental import pallas as pl
from jax.experimental.pallas import tpu as pltpu
from jax.experimental.pallas import tpu_sc as plsc

VOCAB = 1000000
D = 64
PMAX = 200
B = 4096
S = 200
NC, NS = 2, 16
NW = NC * NS
BB = B // 128
UNITS = S * BB
UNITS_W = UNITS // NW
LANES = 16


def _body(seqT_hbm, pidT_hbm, tok_hbm, pos_hbm, out_hbm,
          idx_v, pidx_v, buf, tblock, pos_sp, sem_t, sem_p):
    c = lax.axis_index("c")
    sub = lax.axis_index("s")
    wid = sub * NC + c

    @pl.when(sub == 0)
    def _stage():
        pltpu.sync_copy(pos_hbm, pos_sp)

    plsc.subcore_barrier()

    lane = lax.iota(jnp.int32, LANES)

    def unit_body(u, carry):
        s_idx = u // BB
        bb = u % BB
        pltpu.sync_copy(seqT_hbm.at[s_idx, pl.ds(bb * 128, 128)], idx_v)
        pltpu.sync_copy(pidT_hbm.at[s_idx, pl.ds(bb * 128, 128)], pidx_v)
        pltpu.async_copy(pos_sp.at[pidx_v], buf, sem_p).wait()
        pltpu.async_copy(tok_hbm.at[idx_v], buf, sem_t, add=True).wait()

        def transpose_body(d, acc):
            db = d // 8
            d8 = d % 8
            col = jnp.full((LANES,), d, jnp.int32)
            for g in range(128 // LANES):
                v = plsc.load_gather(buf, [lane + (g * LANES), col])
                tblock[db, d8, pl.ds(g * LANES, LANES)] = v
            return acc

        lax.fori_loop(0, D, transpose_body, 0, unroll=False)
        pltpu.sync_copy(tblock, out_hbm.at[s_idx, :, bb])
        return carry

    lax.fori_loop(wid * UNITS_W, (wid + 1) * UNITS_W, unit_body, 0,
                  unroll=False)


@jax.jit
def _embed_sum(seqT, pidT, token_table, pos_table):
    mesh = plsc.VectorSubcoreMesh(core_axis_name="c", subcore_axis_name="s")
    kern = pl.kernel(
        _body,
        out_type=jax.ShapeDtypeStruct((S, 8, BB, 8, 128), jnp.float32),
        mesh=mesh,
        scratch_types=[
            pltpu.VMEM((128,), jnp.int32),
            pltpu.VMEM((128,), jnp.int32),
            pltpu.VMEM((128, D), jnp.float32),
            pltpu.VMEM((8, 8, 128), jnp.float32),
            pltpu.VMEM_SHARED((PMAX, D), jnp.float32),
            pltpu.SemaphoreType.DMA,
            pltpu.SemaphoreType.DMA,
        ],
        compiler_params=pltpu.CompilerParams(use_tc_tiling_on_sc=False,
                                             needs_layout_passes=False),
    )
    return kern(seqT, pidT, token_table, pos_table)


def kernel(sequence, position_ids, token_table, pos_table):
    seqT = sequence.T.astype(jnp.int32)
    pidT = position_ids.T.astype(jnp.int32)
    w = _embed_sum(seqT, pidT, token_table, pos_table)
    x = w.transpose(0, 1, 3, 2, 4).reshape(S, D, B)
    return x.transpose(2, 0, 1)


# speedup vs baseline: 1.8053x; 1.8053x over previous
import jax
import jax.numpy as jnp
from jax import lax
from jax.experimental import pallas as pl
from jax.experimental.pallas import tpu as pltpu
from jax.experimental.pallas import tpu_sc as plsc

VOCAB = 1000000
D = 64
PMAX = 200
B = 4096
S = 200
NC, NS = 2, 16
NW = NC * NS
BB = B // 128
UNITS = S * BB
UNITS_W = UNITS // NW
LANES = 16
DB = D // 8


def _unit(u):
    return u // BB, u % BB


def _body(seqT_hbm, pidT_hbm, tok_hbm, pos_hbm, out_hbm,
          idx_a, pidx_a, idx_b, pidx_b, buf_a, buf_b, tb_a, tb_b, pos_sp,
          sem_ta, sem_tb, sem_p, sem_w):
    c = lax.axis_index("c")
    sub = lax.axis_index("s")
    wid = sub * NC + c
    base = wid * UNITS_W

    @pl.when(sub == 0)
    def _stage():
        pltpu.sync_copy(pos_hbm, pos_sp)

    plsc.subcore_barrier()

    lane = lax.iota(jnp.int32, LANES)

    def start_unit(u, idx_v, pidx_v, buf, sem_t):
        s_idx, bb = _unit(u)
        pltpu.sync_copy(seqT_hbm.at[s_idx, pl.ds(bb * 128, 128)], idx_v)
        pltpu.sync_copy(pidT_hbm.at[s_idx, pl.ds(bb * 128, 128)], pidx_v)
        pltpu.async_copy(pos_sp.at[pidx_v], buf, sem_p).wait()
        pltpu.async_copy(tok_hbm.at[idx_v], buf, sem_t, add=True)

    def wait_tok(idx_v, buf, sem_t):
        pltpu.make_async_copy(tok_hbm.at[idx_v], buf, sem_t).wait()

    def drain_writes(u, tb):
        s_idx, bb = _unit(u)
        for db in range(DB):
            pltpu.make_async_copy(tb.at[pl.ds(db * 8, 8)],
                                  out_hbm.at[s_idx, db, bb], sem_w).wait()

    def transpose(buf, tb):
        def block_body(m, acc):
            g16 = (m % 8) * LANES
            j16 = (m // 8) * LANES
            row = g16 + lane
            for k in range(LANES):
                rot = (lane + k) & (LANES - 1)
                col = j16 + rot
                v = plsc.load_gather(buf, [row, col])
                plsc.store_scatter(tb, [col, row], v)
            return acc

        lax.fori_loop(0, 32, block_body, 0, unroll=False)

    def write_unit(u, tb):
        s_idx, bb = _unit(u)
        for db in range(DB):
            pltpu.async_copy(tb.at[pl.ds(db * 8, 8)],
                             out_hbm.at[s_idx, db, bb], sem_w)

    start_unit(base, idx_a, pidx_a, buf_a, sem_ta)

    def pair_body(i, carry):
        u_a = base + 2 * i

        @pl.when(u_a + 1 < base + UNITS_W)
        def _sb():
            start_unit(u_a + 1, idx_b, pidx_b, buf_b, sem_tb)
        wait_tok(idx_a, buf_a, sem_ta)

        @pl.when(i >= 1)
        def _da():
            drain_writes(u_a - 2, tb_a)
        transpose(buf_a, tb_a)
        write_unit(u_a, tb_a)

        @pl.when(u_a + 2 < base + UNITS_W)
        def _sa():
            start_unit(u_a + 2, idx_a, pidx_a, buf_a, sem_ta)
        wait_tok(idx_b, buf_b, sem_tb)

        @pl.when(i >= 1)
        def _db():
            drain_writes(u_a - 1, tb_b)
        transpose(buf_b, tb_b)
        write_unit(u_a + 1, tb_b)
        return carry

    lax.fori_loop(0, UNITS_W // 2, pair_body, 0, unroll=False)
    drain_writes(base + UNITS_W - 2, tb_a)
    drain_writes(base + UNITS_W - 1, tb_b)


@jax.jit
def _embed_sum(seqT, pidT, token_table, pos_table):
    mesh = plsc.VectorSubcoreMesh(core_axis_name="c", subcore_axis_name="s")
    kern = pl.kernel(
        _body,
        out_type=jax.ShapeDtypeStruct((S, DB, BB, 8, 128), jnp.float32),
        mesh=mesh,
        scratch_types=[
            pltpu.VMEM((128,), jnp.int32),
            pltpu.VMEM((128,), jnp.int32),
            pltpu.VMEM((128,), jnp.int32),
            pltpu.VMEM((128,), jnp.int32),
            pltpu.VMEM((128, D), jnp.float32),
            pltpu.VMEM((128, D), jnp.float32),
            pltpu.VMEM((D, 128), jnp.float32),
            pltpu.VMEM((D, 128), jnp.float32),
            pltpu.VMEM_SHARED((PMAX, D), jnp.float32),
            pltpu.SemaphoreType.DMA,
            pltpu.SemaphoreType.DMA,
            pltpu.SemaphoreType.DMA,
            pltpu.SemaphoreType.DMA,
        ],
        compiler_params=pltpu.CompilerParams(use_tc_tiling_on_sc=False,
                                             needs_layout_passes=False),
    )
    return kern(seqT, pidT, token_table, pos_table)


def kernel(sequence, position_ids, token_table, pos_table):
    seqT = sequence.T.astype(jnp.int32)
    pidT = position_ids.T.astype(jnp.int32)
    w = _embed_sum(seqT, pidT, token_table, pos_table)
    x = w.transpose(0, 1, 3, 2, 4).reshape(S, D, B)
    return x.transpose(2, 0, 1)


# speedup vs baseline: 1.9128x; 1.0596x over previous
import jax
import jax.numpy as jnp
from jax import lax
from jax.experimental import pallas as pl
from jax.experimental.pallas import tpu as pltpu
from jax.experimental.pallas import tpu_sc as plsc

VOCAB = 1000000
D = 64
PMAX = 200
B = 4096
S = 200
NC, NS = 2, 16
NW = NC * NS
BB = B // 128
UNITS = S * BB
UNITS_W = UNITS // NW
LANES = 16
DB = D // 8


def _unit(u):
    return u // BB, u % BB


def _body(seqT_hbm, pidT_hbm, tok_hbm, pos_hbm, out_hbm,
          idx_a, pidx_a, idx_b, pidx_b, buf_a, buf_b, tb_a, tb_b, pos_sp,
          sem_ta, sem_tb, sem_pa, sem_pb, sem_wa, sem_wb):
    c = lax.axis_index("c")
    sub = lax.axis_index("s")
    wid = sub * NC + c
    base = wid * UNITS_W
    end = base + UNITS_W

    @pl.when(sub == 0)
    def _stage():
        pltpu.sync_copy(pos_hbm, pos_sp)

    plsc.subcore_barrier()

    lane = lax.iota(jnp.int32, LANES)
    rotv = [(lane + k) & (LANES - 1) for k in range(LANES)]
    stv = [rotv[k] * 128 + lane for k in range(LANES)]

    def load_idx(u, idx_v, pidx_v, buf, sem_p):
        s_idx, bb = _unit(u)
        pltpu.sync_copy(seqT_hbm.at[s_idx, pl.ds(bb * 128, 128)], idx_v)
        pltpu.sync_copy(pidT_hbm.at[s_idx, pl.ds(bb * 128, 128)], pidx_v)
        pltpu.async_copy(pos_sp.at[pidx_v], buf, sem_p)

    def start_tok(idx_v, pidx_v, buf, sem_p, sem_t):
        pltpu.make_async_copy(pos_sp.at[pidx_v], buf, sem_p).wait()
        pltpu.async_copy(tok_hbm.at[idx_v], buf, sem_t, add=True)

    def wait_tok(idx_v, buf, sem_t):
        pltpu.make_async_copy(tok_hbm.at[idx_v], buf, sem_t).wait()

    def transpose(buf, tb):
        def block_body(m, acc):
            g16 = (m % 8) * LANES
            j16 = (m // 8) * LANES
            row = g16 + lane
            sb = j16 * 128 + g16
            for k in range(LANES):
                v = plsc.load_gather(buf, [row, rotv[k] + j16])
                plsc.store_scatter(tb, [stv[k] + sb], v)
            return acc

        lax.fori_loop(0, 32, block_body, 0, unroll=False)

    def drain_writes(u, tb, sem_w):
        s_idx, bb = _unit(u)
        for db in range(DB):
            pltpu.make_async_copy(tb.at[pl.ds(db * 1024, 1024)],
                                  out_hbm.at[s_idx, db, bb], sem_w).wait()

    def write_unit(u, tb, sem_w):
        s_idx, bb = _unit(u)
        for db in range(DB):
            pltpu.async_copy(tb.at[pl.ds(db * 1024, 1024)],
                             out_hbm.at[s_idx, db, bb], sem_w)

    slot_a = (idx_a, pidx_a, buf_a, tb_a, sem_ta, sem_pa, sem_wa)
    slot_b = (idx_b, pidx_b, buf_b, tb_b, sem_tb, sem_pb, sem_wb)

    load_idx(base, idx_a, pidx_a, buf_a, sem_pa)
    start_tok(idx_a, pidx_a, buf_a, sem_pa, sem_ta)
    load_idx(base + 1, idx_b, pidx_b, buf_b, sem_pb)

    def phase(u, i, cur, nxt):
        idx_c, pidx_c, buf_c, tb_c, sem_tc, sem_pc, sem_wc = cur
        idx_n, pidx_n, buf_n, tb_n, sem_tn, sem_pn, sem_wn = nxt
        wait_tok(idx_c, buf_c, sem_tc)

        @pl.when(u + 1 < end)
        def _tok_next():
            start_tok(idx_n, pidx_n, buf_n, sem_pn, sem_tn)

        @pl.when(i >= 1)
        def _drain():
            drain_writes(u - 2, tb_c, sem_wc)
        transpose(buf_c, tb_c)

        @pl.when(u + 2 < end)
        def _stage_next():
            load_idx(u + 2, idx_c, pidx_c, buf_c, sem_pc)
        write_unit(u, tb_c, sem_wc)

    def pair_body(i, carry):
        u_a = base + 2 * i
        phase(u_a, i, slot_a, slot_b)
        phase(u_a + 1, i, slot_b, slot_a)
        return carry

    lax.fori_loop(0, UNITS_W // 2, pair_body, 0, unroll=False)
    drain_writes(end - 2, tb_a, sem_wa)
    drain_writes(end - 1, tb_b, sem_wb)


@jax.jit
def _embed_sum(seqT, pidT, token_table, pos_table):
    mesh = plsc.VectorSubcoreMesh(core_axis_name="c", subcore_axis_name="s")
    kern = pl.kernel(
        _body,
        out_type=jax.ShapeDtypeStruct((S, DB, BB, 1024), jnp.float32),
        mesh=mesh,
        scratch_types=[
            pltpu.VMEM((128,), jnp.int32),
            pltpu.VMEM((128,), jnp.int32),
            pltpu.VMEM((128,), jnp.int32),
            pltpu.VMEM((128,), jnp.int32),
            pltpu.VMEM((128, D), jnp.float32),
            pltpu.VMEM((128, D), jnp.float32),
            pltpu.VMEM((D * 128,), jnp.float32),
            pltpu.VMEM((D * 128,), jnp.float32),
            pltpu.VMEM_SHARED((PMAX, D), jnp.float32),
            pltpu.SemaphoreType.DMA,
            pltpu.SemaphoreType.DMA,
            pltpu.SemaphoreType.DMA,
            pltpu.SemaphoreType.DMA,
            pltpu.SemaphoreType.DMA,
            pltpu.SemaphoreType.DMA,
        ],
        compiler_params=pltpu.CompilerParams(use_tc_tiling_on_sc=False,
                                             needs_layout_passes=False),
    )
    return kern(seqT, pidT, token_table, pos_table)


def kernel(sequence, position_ids, token_table, pos_table):
    tok2 = jnp.pad(token_table, ((0, 0), (0, D))).reshape(2 * VOCAB, D)
    seqT = (sequence.T * 2).astype(jnp.int32)
    pidT = position_ids.T.astype(jnp.int32)
    w = _embed_sum(seqT, pidT, tok2, pos_table)
    x = w.reshape(S, DB, BB, 8, 128).transpose(0, 1, 3, 2, 4).reshape(S, D, B)
    return x.transpose(2, 0, 1)


# speedup vs baseline: 2.0810x; 1.0879x over previous
import jax
import jax.numpy as jnp
from jax import lax
from jax.experimental import pallas as pl
from jax.experimental.pallas import tpu as pltpu
from jax.experimental.pallas import tpu_sc as plsc

VOCAB = 1000000
D = 64
PMAX = 200
B = 4096
S = 200
NC, NS = 2, 16
NW = NC * NS
T = 2
G = B // (128 * T)
UNITS = S * G
UNITS_W = UNITS // NW
LANES = 16
DB = D // 8


def _unit(u):
    return u // G, u % G


def _body(seqT_hbm, pidT_hbm, tok_hbm, pos_hbm, out_hbm,
          idx_a, pidx_a, idx_b, pidx_b, buf_a, buf_b, tb_a, tb_b, pos_sp,
          sem_ta, sem_tb, sem_pa, sem_pb, sem_wa, sem_wb):
    c = lax.axis_index("c")
    sub = lax.axis_index("s")
    wid = sub * NC + c
    base = wid * UNITS_W
    end = base + UNITS_W

    @pl.when(sub == 0)
    def _stage():
        pltpu.sync_copy(pos_hbm, pos_sp)

    plsc.subcore_barrier()

    lane = lax.iota(jnp.int32, LANES)
    rotv = [(lane + k) & (LANES - 1) for k in range(LANES)]
    stv = [rotv[k] * 128 + lane for k in range(LANES)]

    def load_idx(u, idx_v, pidx_v, buf, sem_p):
        s_idx, g = _unit(u)
        pltpu.sync_copy(seqT_hbm.at[s_idx, g], idx_v)
        pltpu.sync_copy(pidT_hbm.at[s_idx, g], pidx_v)
        for t in range(T):
            pltpu.async_copy(pos_sp.at[pidx_v.at[t]], buf.at[t], sem_p)

    def start_tok(idx_v, pidx_v, buf, sem_p, sem_t):
        for t in range(T):
            pltpu.make_async_copy(pos_sp.at[pidx_v.at[t]], buf.at[t],
                                  sem_p).wait()
        for t in range(T):
            pltpu.async_copy(tok_hbm.at[idx_v.at[t]], buf.at[t], sem_t,
                             add=True)

    def wait_tok(idx_v, buf, sem_t):
        for t in range(T):
            pltpu.make_async_copy(tok_hbm.at[idx_v.at[t]], buf.at[t],
                                  sem_t).wait()

    def transpose(buf, tb):
        for t in range(T):
            bt = buf.at[t]
            tt = tb.at[t]

            def block_body(m, acc):
                g16 = (m % 8) * LANES
                j16 = (m // 8) * LANES
                row = g16 + lane
                sb = j16 * 128 + g16
                for k in range(LANES):
                    v = plsc.load_gather(bt, [row, rotv[k] + j16])
                    plsc.store_scatter(tt, [stv[k] + sb], v)
                return acc

            lax.fori_loop(0, 32, block_body, 0, unroll=False)

    def drain_writes(u, tb, sem_w):
        s_idx, g = _unit(u)
        for db in range(DB):
            pltpu.make_async_copy(tb.at[:, pl.ds(db * 1024, 1024)],
                                  out_hbm.at[s_idx, db, g], sem_w).wait()

    def write_unit(u, tb, sem_w):
        s_idx, g = _unit(u)
        for db in range(DB):
            pltpu.async_copy(tb.at[:, pl.ds(db * 1024, 1024)],
                             out_hbm.at[s_idx, db, g], sem_w)

    slot_a = (idx_a, pidx_a, buf_a, tb_a, sem_ta, sem_pa, sem_wa)
    slot_b = (idx_b, pidx_b, buf_b, tb_b, sem_tb, sem_pb, sem_wb)

    load_idx(base, idx_a, pidx_a, buf_a, sem_pa)
    start_tok(idx_a, pidx_a, buf_a, sem_pa, sem_ta)
    load_idx(base + 1, idx_b, pidx_b, buf_b, sem_pb)

    def phase(u, i, cur, nxt):
        idx_c, pidx_c, buf_c, tb_c, sem_tc, sem_pc, sem_wc = cur
        idx_n, pidx_n, buf_n, tb_n, sem_tn, sem_pn, sem_wn = nxt
        wait_tok(idx_c, buf_c, sem_tc)

        @pl.when(u + 1 < end)
        def _tok_next():
            start_tok(idx_n, pidx_n, buf_n, sem_pn, sem_tn)

        @pl.when(i >= 1)
        def _drain():
            drain_writes(u - 2, tb_c, sem_wc)
        transpose(buf_c, tb_c)

        @pl.when(u + 2 < end)
        def _stage_next():
            load_idx(u + 2, idx_c, pidx_c, buf_c, sem_pc)
        write_unit(u, tb_c, sem_wc)

    def pair_body(i, carry):
        u_a = base + 2 * i
        phase(u_a, i, slot_a, slot_b)
        phase(u_a + 1, i, slot_b, slot_a)
        return carry

    lax.fori_loop(0, UNITS_W // 2, pair_body, 0, unroll=False)
    drain_writes(end - 2, tb_a, sem_wa)
    drain_writes(end - 1, tb_b, sem_wb)


@jax.jit
def _embed_sum(seqT, pidT, token_table, pos_table):
    mesh = plsc.VectorSubcoreMesh(core_axis_name="c", subcore_axis_name="s")
    kern = pl.kernel(
        _body,
        out_type=jax.ShapeDtypeStruct((S, DB, G, T, 1024), jnp.float32),
        mesh=mesh,
        scratch_types=[
            pltpu.VMEM((T, 128), jnp.int32),
            pltpu.VMEM((T, 128), jnp.int32),
            pltpu.VMEM((T, 128), jnp.int32),
            pltpu.VMEM((T, 128), jnp.int32),
            pltpu.VMEM((T, 128, D), jnp.float32),
            pltpu.VMEM((T, 128, D), jnp.float32),
            pltpu.VMEM((T, D * 128), jnp.float32),
            pltpu.VMEM((T, D * 128), jnp.float32),
            pltpu.VMEM_SHARED((PMAX, D), jnp.float32),
            pltpu.SemaphoreType.DMA,
            pltpu.SemaphoreType.DMA,
            pltpu.SemaphoreType.DMA,
            pltpu.SemaphoreType.DMA,
            pltpu.SemaphoreType.DMA,
            pltpu.SemaphoreType.DMA,
        ],
        compiler_params=pltpu.CompilerParams(use_tc_tiling_on_sc=False,
                                             needs_layout_passes=False),
    )
    return kern(seqT, pidT, token_table, pos_table)


def kernel(sequence, position_ids, token_table, pos_table):
    tok2 = jnp.pad(token_table, ((0, 0), (0, D))).reshape(2 * VOCAB, D)
    seqT = (sequence.T * 2).astype(jnp.int32).reshape(S, G, T, 128)
    pidT = position_ids.T.astype(jnp.int32).reshape(S, G, T, 128)
    w = _embed_sum(seqT, pidT, tok2, pos_table)
    x = (w.reshape(S, DB, G, T, 8, 128)
          .transpose(0, 1, 4, 2, 3, 5)
          .reshape(S, D, B))
    return x.transpose(2, 0, 1)


# speedup vs baseline: 2.1718x; 1.0436x over previous
import jax
import jax.numpy as jnp
from jax import lax
from jax.experimental import pallas as pl
from jax.experimental.pallas import tpu as pltpu
from jax.experimental.pallas import tpu_sc as plsc

VOCAB = 1000000
D = 64
PMAX = 200
B = 4096
S = 200
NC, NS = 2, 16
NW = NC * NS
T = 4
G = B // (128 * T)
UNITS = S * G
UNITS_W = UNITS // NW
LANES = 16
DB = D // 8


def _unit(u):
    return u // G, u % G


def _body(seqT_hbm, pidT_hbm, tok_hbm, pos_hbm, out_hbm,
          idx_a, pidx_a, idx_b, pidx_b, buf_a, buf_b, tb, pos_sp,
          sem_ta, sem_tb, sem_pa, sem_pb, sem_w):
    c = lax.axis_index("c")
    sub = lax.axis_index("s")
    wid = sub * NC + c
    base = wid * UNITS_W
    end = base + UNITS_W

    @pl.when(sub == 0)
    def _stage():
        pltpu.sync_copy(pos_hbm, pos_sp)

    plsc.subcore_barrier()

    lane = lax.iota(jnp.int32, LANES)
    rotv = [(lane + k) & (LANES - 1) for k in range(LANES)]
    stv = [rotv[k] * 128 + lane for k in range(LANES)]

    def load_idx(u, idx_v, pidx_v, buf, sem_p):
        s_idx, g = _unit(u)
        pltpu.sync_copy(seqT_hbm.at[s_idx, g], idx_v)
        pltpu.sync_copy(pidT_hbm.at[s_idx, g], pidx_v)
        for t in range(T):
            pltpu.async_copy(pos_sp.at[pidx_v.at[t]], buf.at[t], sem_p)

    def start_tok(idx_v, pidx_v, buf, sem_p, sem_t):
        for t in range(T):
            pltpu.make_async_copy(pos_sp.at[pidx_v.at[t]], buf.at[t],
                                  sem_p).wait()
        for t in range(T):
            pltpu.async_copy(tok_hbm.at[idx_v.at[t]], buf.at[t], sem_t,
                             add=True)

    def wait_tok(idx_v, buf, sem_t):
        for t in range(T):
            pltpu.make_async_copy(tok_hbm.at[idx_v.at[t]], buf.at[t],
                                  sem_t).wait()

    def transpose(buf, tb):
        for t in range(T):
            bt = buf.at[t]
            tt = tb.at[t]

            def block_body(m, acc):
                g16 = (m % 8) * LANES
                j16 = (m // 8) * LANES
                row = g16 + lane
                sb = j16 * 128 + g16
                for k in range(LANES):
                    v = plsc.load_gather(bt, [row, rotv[k] + j16])
                    plsc.store_scatter(tt, [stv[k] + sb], v)
                return acc

            lax.fori_loop(0, 32, block_body, 0, unroll=False)

    def drain_writes(u, tb, sem_w):
        s_idx, g = _unit(u)
        for db in range(DB):
            pltpu.make_async_copy(tb.at[:, pl.ds(db * 1024, 1024)],
                                  out_hbm.at[s_idx, db, g], sem_w).wait()

    def write_unit(u, tb, sem_w):
        s_idx, g = _unit(u)
        for db in range(DB):
            pltpu.async_copy(tb.at[:, pl.ds(db * 1024, 1024)],
                             out_hbm.at[s_idx, db, g], sem_w)

    slot_a = (idx_a, pidx_a, buf_a, sem_ta, sem_pa)
    slot_b = (idx_b, pidx_b, buf_b, sem_tb, sem_pb)

    load_idx(base, idx_a, pidx_a, buf_a, sem_pa)
    start_tok(idx_a, pidx_a, buf_a, sem_pa, sem_ta)
    load_idx(base + 1, idx_b, pidx_b, buf_b, sem_pb)

    def phase(u, cur, nxt):
        idx_c, pidx_c, buf_c, sem_tc, sem_pc = cur
        idx_n, pidx_n, buf_n, sem_tn, sem_pn = nxt
        wait_tok(idx_c, buf_c, sem_tc)

        @pl.when(u + 1 < end)
        def _tok_next():
            start_tok(idx_n, pidx_n, buf_n, sem_pn, sem_tn)

        @pl.when(u >= base + 1)
        def _drain():
            drain_writes(u - 1, tb, sem_w)
        transpose(buf_c, tb)

        @pl.when(u + 2 < end)
        def _stage_next():
            load_idx(u + 2, idx_c, pidx_c, buf_c, sem_pc)
        write_unit(u, tb, sem_w)

    def pair_body(i, carry):
        u_a = base + 2 * i
        phase(u_a, slot_a, slot_b)
        phase(u_a + 1, slot_b, slot_a)
        return carry

    lax.fori_loop(0, UNITS_W // 2, pair_body, 0, unroll=False)
    drain_writes(end - 1, tb, sem_w)


@jax.jit
def _embed_sum(seqT, pidT, token_table, pos_table):
    mesh = plsc.VectorSubcoreMesh(core_axis_name="c", subcore_axis_name="s")
    kern = pl.kernel(
        _body,
        out_type=jax.ShapeDtypeStruct((S, DB, G, T, 1024), jnp.float32),
        mesh=mesh,
        scratch_types=[
            pltpu.VMEM((T, 128), jnp.int32),
            pltpu.VMEM((T, 128), jnp.int32),
            pltpu.VMEM((T, 128), jnp.int32),
            pltpu.VMEM((T, 128), jnp.int32),
            pltpu.VMEM((T, 128, D), jnp.float32),
            pltpu.VMEM((T, 128, D), jnp.float32),
            pltpu.VMEM((T, D * 128), jnp.float32),
            pltpu.VMEM_SHARED((PMAX, D), jnp.float32),
            pltpu.SemaphoreType.DMA,
            pltpu.SemaphoreType.DMA,
            pltpu.SemaphoreType.DMA,
            pltpu.SemaphoreType.DMA,
            pltpu.SemaphoreType.DMA,
        ],
        compiler_params=pltpu.CompilerParams(use_tc_tiling_on_sc=False,
                                             needs_layout_passes=False),
    )
    return kern(seqT, pidT, token_table, pos_table)


def kernel(sequence, position_ids, token_table, pos_table):
    tok2 = jnp.pad(token_table, ((0, 0), (0, D))).reshape(2 * VOCAB, D)
    seqT = (sequence.T * 2).astype(jnp.int32).reshape(S, G, T, 128)
    pidT = position_ids.T.astype(jnp.int32).reshape(S, G, T, 128)
    w = _embed_sum(seqT, pidT, tok2, pos_table)
    x = (w.reshape(S, DB, G, T, 8, 128)
          .transpose(0, 1, 4, 2, 3, 5)
          .reshape(S, D, B))
    return x.transpose(2, 0, 1)
